# baseline (speedup 1.0000x reference)
import functools

import jax
import jax.numpy as jnp
from jax import lax
from jax.experimental import pallas as pl
from jax.experimental.pallas import tpu as pltpu
from jax.experimental.pallas import tpu_sc as plsc

VOCAB = 100000
DIM = 128
L = 20
B = 4096
S = 1024

NC = 2
NS = 16
NW = NC * NS
B_PER_W = B // NW
D_PER_W = DIM // NW
S_CHUNKS = S // 128


def _sc_body(x_hbm, samp_hbm, emb_hbm, wflat_hbm,
             s_hbm, cols_hbm,
             idx_v, ebuf_v, acc_v, sidx_v, widx_v, colbuf_v, sem):
    wid = lax.axis_index("s") * NC + lax.axis_index("c")
    base = wid * B_PER_W

    for l in range(L):
        pltpu.sync_copy(x_hbm.at[pl.ds(l * B + base, B_PER_W)], idx_v.at[l])

    for l in range(L):
        pltpu.async_copy(emb_hbm.at[idx_v.at[l]], ebuf_v, sem).wait()

        def acc_row(i, _, first=(l == 0)):
            for c in range(DIM // 16):
                sl = pl.ds(c * 16, 16)
                if first:
                    acc_v[i, sl] = ebuf_v[i, sl]
                else:
                    acc_v[i, sl] = acc_v[i, sl] + ebuf_v[i, sl]
            return 0

        lax.fori_loop(0, B_PER_W, acc_row, 0)

    pltpu.sync_copy(acc_v, s_hbm.at[pl.ds(base, B_PER_W)])

    for j in range(S_CHUNKS):
        pltpu.sync_copy(samp_hbm.at[pl.ds(j * 128, 128)], sidx_v.at[j])
    d0 = wid * D_PER_W
    for d in range(D_PER_W):
        def make_widx(j, _, d=d):
            off = (d0 + d) * VOCAB
            for c in range(8):
                sl = pl.ds(c * 16, 16)
                widx_v[j, sl] = sidx_v[j, sl] + off
            return 0

        lax.fori_loop(0, S_CHUNKS, make_widx, 0)
        for j in range(S_CHUNKS):
            pltpu.async_copy(wflat_hbm.at[widx_v.at[j]],
                             colbuf_v.at[d, j], sem).wait()
    pltpu.sync_copy(colbuf_v, cols_hbm.at[pl.ds(d0, D_PER_W)])


@jax.jit
def _sc_call(x_flat, samp, emb, wflat):
    mesh = plsc.VectorSubcoreMesh(core_axis_name="c", subcore_axis_name="s")
    return pl.kernel(
        _sc_body,
        out_type=(
            jax.ShapeDtypeStruct((B, DIM), jnp.float32),
            jax.ShapeDtypeStruct((DIM, S_CHUNKS, 128), jnp.float32),
        ),
        mesh=mesh,
        scratch_types=[
            pltpu.VMEM((L, B_PER_W), jnp.int32),
            pltpu.VMEM((B_PER_W, DIM), jnp.float32),
            pltpu.VMEM((B_PER_W, DIM), jnp.float32),
            pltpu.VMEM((S_CHUNKS, 128), jnp.int32),
            pltpu.VMEM((S_CHUNKS, 128), jnp.int32),
            pltpu.VMEM((D_PER_W, S_CHUNKS, 128), jnp.float32),
            pltpu.SemaphoreType.DMA,
        ],
    )(x_flat, samp, emb, wflat)


def _tc_body(s_ref, cols_ref, out_ref):
    logits = jnp.dot(s_ref[...], cols_ref[...],
                     preferred_element_type=jnp.float32)
    out_ref[...] = 1.0 / (1.0 + jnp.exp(-logits))


@jax.jit
def _tc_call(s, cols):
    bt = 512
    return pl.pallas_call(
        _tc_body,
        grid=(B // bt,),
        in_specs=[
            pl.BlockSpec((bt, DIM), lambda i: (i, 0)),
            pl.BlockSpec((DIM, S), lambda i: (0, 0)),
        ],
        out_specs=pl.BlockSpec((bt, S), lambda i: (i, 0)),
        out_shape=jax.ShapeDtypeStruct((B, S), jnp.float32),
    )(s, cols)


def kernel(x, sample, emb, W):
    x_flat = x.astype(jnp.int32).reshape(-1)
    samp = sample.astype(jnp.int32)
    wflat = W.reshape(-1)
    s, cols3 = _sc_call(x_flat, samp, emb, wflat)
    return _tc_call(s, cols3.reshape(DIM, S))


# speedup vs baseline: 4.1310x; 4.1310x over previous
import jax
import jax.numpy as jnp
from jax import lax
from jax.experimental import pallas as pl
from jax.experimental.pallas import tpu as pltpu
from jax.experimental.pallas import tpu_sc as plsc

VOCAB = 100000
DIM = 128
L = 20
B = 4096
S = 1024

NC = 2
NS = 16
NW = NC * NS
B_PER_W = B // NW
S_PER_W = S // NW
CH = 16
NCHUNK = B_PER_W // CH
ROWS = CH * L
PIECES = ((0, 320),)


def _sc_body(xt_hbm, samp_hbm, emb_hbm, wt_hbm,
             s_hbm, colst_hbm,
             idx_v, gbuf0_v, gbuf1_v, sbuf0_v, sbuf1_v, sidx_v, cbuf_v,
             sem0, sem1, csem, wsem0, wsem1):
    wid = lax.axis_index("s") * NC + lax.axis_index("c")
    base = wid * B_PER_W

    pltpu.sync_copy(xt_hbm.at[pl.ds(base * L, B_PER_W * L)], idx_v)

    def fire(ch, gbuf, sem):
        for (o, n) in PIECES:
            pltpu.async_copy(
                emb_hbm.at[idx_v.at[pl.ds(ch * ROWS + o, n)]],
                gbuf.at[pl.ds(o, n)], sem)

    def drain_gather(gbuf, sem):
        pltpu.make_async_copy(emb_hbm.at[pl.ds(0, ROWS)], gbuf, sem).wait()

    def drain_write(sbuf, wsem):
        pltpu.make_async_copy(sbuf, s_hbm.at[pl.ds(0, CH)], wsem).wait()

    fire(0, gbuf0_v, sem0)

    sbase = wid * S_PER_W
    pltpu.sync_copy(samp_hbm.at[pl.ds(sbase, S_PER_W)], sidx_v)
    ccopy = pltpu.async_copy(wt_hbm.at[sidx_v], cbuf_v, csem)

    def acc_chunk(ch, gbuf, sbuf, wsem):
        def acc_elem(i, _):
            for c in range(DIM // 16):
                sl = pl.ds(c * 16, 16)
                accs = [gbuf[i * L + j, sl] for j in range(4)]
                for l in range(4, L):
                    accs[l & 3] = accs[l & 3] + gbuf[i * L + l, sl]
                sbuf[i, sl] = (accs[0] + accs[1]) + (accs[2] + accs[3])
            return 0

        lax.fori_loop(0, CH, acc_elem, 0)
        start = pl.multiple_of(base + ch * CH, CH)
        pltpu.async_copy(sbuf, s_hbm.at[pl.ds(start, CH)], wsem)

    def pair(g, _):
        even = 2 * g
        fire(even + 1, gbuf1_v, sem1)
        drain_gather(gbuf0_v, sem0)

        @pl.when(g > 0)
        def _():
            drain_write(sbuf0_v, wsem0)

        acc_chunk(even, gbuf0_v, sbuf0_v, wsem0)

        @pl.when(g < NCHUNK // 2 - 1)
        def _():
            fire(even + 2, gbuf0_v, sem0)

        drain_gather(gbuf1_v, sem1)

        @pl.when(g > 0)
        def _():
            drain_write(sbuf1_v, wsem1)

        acc_chunk(even + 1, gbuf1_v, sbuf1_v, wsem1)
        return 0

    lax.fori_loop(0, NCHUNK // 2, pair, 0)

    drain_write(sbuf0_v, wsem0)
    drain_write(sbuf1_v, wsem1)
    ccopy.wait()
    pltpu.sync_copy(cbuf_v, colst_hbm.at[pl.ds(sbase, S_PER_W)])


@jax.jit
def _sc_call(xt, samp, emb, wt):
    mesh = plsc.VectorSubcoreMesh(core_axis_name="c", subcore_axis_name="s")
    return pl.kernel(
        _sc_body,
        out_type=(
            jax.ShapeDtypeStruct((B, DIM), jnp.float32),
            jax.ShapeDtypeStruct((S, DIM), jnp.float32),
        ),
        mesh=mesh,
        scratch_types=[
            pltpu.VMEM((B_PER_W * L,), jnp.int32),
            pltpu.VMEM((ROWS, DIM), jnp.float32),
            pltpu.VMEM((ROWS, DIM), jnp.float32),
            pltpu.VMEM((CH, DIM), jnp.float32),
            pltpu.VMEM((CH, DIM), jnp.float32),
            pltpu.VMEM((S_PER_W,), jnp.int32),
            pltpu.VMEM((S_PER_W, DIM), jnp.float32),
            pltpu.SemaphoreType.DMA,
            pltpu.SemaphoreType.DMA,
            pltpu.SemaphoreType.DMA,
            pltpu.SemaphoreType.DMA,
            pltpu.SemaphoreType.DMA,
        ],
    )(xt, samp, emb, wt)


def _tc_body(s_ref, colst_ref, out_ref):
    sb = s_ref[...].astype(jnp.bfloat16)
    cb = colst_ref[...].astype(jnp.bfloat16)
    logits = jax.lax.dot_general(
        sb, cb, (((1,), (1,)), ((), ())),
        preferred_element_type=jnp.float32)
    out_ref[...] = 1.0 / (1.0 + jnp.exp(-logits))


@jax.jit
def _tc_call(s, colst):
    bt = 1024
    return pl.pallas_call(
        _tc_body,
        grid=(B // bt,),
        in_specs=[
            pl.BlockSpec((bt, DIM), lambda i: (i, 0)),
            pl.BlockSpec((S, DIM), lambda i: (0, 0)),
        ],
        out_specs=pl.BlockSpec((bt, S), lambda i: (i, 0)),
        out_shape=jax.ShapeDtypeStruct((B, S), jnp.float32),
    )(s, colst)


def kernel(x, sample, emb, W):
    xt = x.astype(jnp.int32).T.reshape(-1)
    samp = sample.astype(jnp.int32)
    wt = W.T
    s, colst = _sc_call(xt, samp, emb, wt)
    return _tc_call(s, colst)
